# baseline (speedup 1.0000x reference)
import functools

import jax
import jax.numpy as jnp
from jax import lax
from jax.experimental import pallas as pl
from jax.experimental.pallas import tpu as pltpu
from jax.experimental.pallas import tpu_sc as plsc

_N = 25000
_E = 150000
_D = 128
_COEF = 0.1
_BN_EPS = 1e-5

_NC = 2
_NT = 16
_CH = 112


@functools.lru_cache(maxsize=None)
def _build_sc_segsum(n, e):
    nch = e // _CH
    assert nch % (2 * _NT) == 0
    nk = nch // _NT
    q = ((n + _NC * _NT - 1) // (_NC * _NT) + 7) // 8 * 8
    split = _NT * q
    trash = split
    acc_rows = split + _NT
    last = n - split - (_NT - 1) * q
    assert 0 < last <= q and split <= n

    mesh = plsc.VectorSubcoreMesh(core_axis_name="c", subcore_axis_name="s")
    f32 = jnp.float32
    i32 = jnp.int32
    osd = jax.ShapeDtypeStruct((n, _D), f32)
    qcap = 2 * _CH + 16

    @functools.partial(
        pl.kernel,
        out_type=(osd, osd, osd),
        mesh=mesh,
        scratch_types=[
            pltpu.VMEM_SHARED((acc_rows, _D), f32),
            [pltpu.VMEM((_CH,), i32)] * 2,
            [pltpu.VMEM((_CH,), i32)] * 2,
            pltpu.VMEM((qcap,), i32),
            pltpu.VMEM((qcap,), i32),
            [pltpu.VMEM((_CH,), i32)] * 2,
            [pltpu.VMEM((_CH,), i32)] * 2,
            [pltpu.VMEM((_CH, _D), f32)] * 2,
            pltpu.SemaphoreType.DMA,
            pltpu.SemaphoreType.DMA,
            pltpu.SemaphoreType.DMA,
        ],
    )
    def seg(x0, x1, s101, d101, s021, d021, s110, d110, s030, d030,
            out_a, out_b, out_c, acc, src_v, dst_v, qsrc, qdl, fsrc, fdl,
            rows_v, sem_i, sem_g, sem_s):
        c = lax.axis_index("c")
        s = lax.axis_index("s")
        lo = c * split
        hi = jnp.where(c == 0, split, n)
        base = s * q
        trash_s = trash + s
        iota = lax.iota(i32, 16)
        ng = _CH // 16
        _zero16 = jnp.zeros((16,), i32)

        _gdn = lax.GatherDimensionNumbers(offset_dims=(),
                                          collapsed_slice_dims=(0,),
                                          start_index_map=(0,))

        def _lane_gather(x, idx):
            return lax.gather(x, idx[:, None], _gdn, slice_sizes=(1,),
                              mode=lax.GatherScatterMode.PROMISE_IN_BOUNDS)

        def _psum(v):
            x = v
            for sh in (1, 2, 4, 8):
                g = _lane_gather(x, jnp.maximum(iota - sh, 0))
                x = x + jnp.where(iota >= sh, g, _zero16)
            return x

        def _rank_index(cs):
            r = iota + 1
            pos = _zero16
            for w in (8, 4, 2, 1):
                cv = _lane_gather(cs, pos + (w - 1))
                pos = pos + jnp.where(cv < r, jnp.int32(w), jnp.int32(0))
            return pos

        def _scan_edges(xt, st, dt):
            def _issue_idx(k, b):
                off = (s + k * _NT) * _CH
                pltpu.async_copy(st.at[pl.ds(off, _CH)], src_v[b], sem_i)
                pltpu.async_copy(dt.at[pl.ds(off, _CH)], dst_v[b], sem_i)

            def _wait_idx(k, b):
                off = (s + k * _NT) * _CH
                pltpu.make_async_copy(st.at[pl.ds(off, _CH)], src_v[b],
                                      sem_i).wait()
                pltpu.make_async_copy(dt.at[pl.ds(off, _CH)], dst_v[b],
                                      sem_i).wait()

            def _fire(pp, fc):
                op = 1 - pp

                @pl.when(fc >= 1)
                def _():
                    pltpu.make_async_copy(xt.at[fsrc[op]], rows_v[op],
                                          sem_g).wait()
                    pltpu.async_copy(rows_v[op], acc.at[fdl[op]], sem_s,
                                     add=True)

                @pl.when(fc >= 2)
                def _():
                    pltpu.make_async_copy(rows_v[pp], acc.at[fdl[pp]],
                                          sem_s).wait()

                for j in range(ng):
                    fsrc[pp][pl.ds(j * 16, 16)] = qsrc[pl.ds(j * 16, 16)]
                    fdl[pp][pl.ds(j * 16, 16)] = qdl[pl.ds(j * 16, 16)]
                pltpu.async_copy(xt.at[fsrc[pp]], rows_v[pp], sem_g)
                for j in range(ng):
                    qsrc[pl.ds(j * 16, 16)] = qsrc[pl.ds(_CH + j * 16, 16)]
                    qdl[pl.ds(j * 16, 16)] = qdl[pl.ds(_CH + j * 16, 16)]

            def _drain(pp, fc, qc):
                op = 1 - pp

                @pl.when(fc >= 1)
                def _():
                    pltpu.make_async_copy(xt.at[fsrc[op]], rows_v[op],
                                          sem_g).wait()
                    pltpu.async_copy(rows_v[op], acc.at[fdl[op]], sem_s,
                                     add=True)

                @pl.when(fc >= 2)
                def _():
                    pltpu.make_async_copy(rows_v[pp], acc.at[fdl[pp]],
                                          sem_s).wait()

                for j in range(ng):
                    m = (j * 16 + iota) < qc
                    fsrc[pp][pl.ds(j * 16, 16)] = jnp.where(
                        m, qsrc[pl.ds(j * 16, 16)], 0)
                    fdl[pp][pl.ds(j * 16, 16)] = jnp.where(
                        m, qdl[pl.ds(j * 16, 16)], trash_s)

                pltpu.async_copy(xt.at[fsrc[pp]], rows_v[pp], sem_g)
                pltpu.make_async_copy(xt.at[fsrc[pp]], rows_v[pp],
                                      sem_g).wait()
                pltpu.async_copy(rows_v[pp], acc.at[fdl[pp]], sem_s,
                                 add=True)

                @pl.when(fc >= 1)
                def _():
                    pltpu.make_async_copy(rows_v[op], acc.at[fdl[op]],
                                          sem_s).wait()

                pltpu.make_async_copy(rows_v[pp], acc.at[fdl[pp]],
                                      sem_s).wait()

            def _chunk(k, b, qn, lc, fc, lv_src, lv_dl):
                @pl.when(k + 1 < nk)
                def _():
                    _issue_idx(k + 1, 1 - b)

                _wait_idx(k, b)
                for j in range(ng):
                    d = dst_v[b][pl.ds(j * 16, 16)]
                    sr = src_v[b][pl.ds(j * 16, 16)]
                    ok = (d >= lo) & (d < hi)
                    oki = jnp.where(ok, jnp.int32(1), jnp.int32(0))
                    cs = _psum(oki)
                    pc = cs[15]
                    inv = _rank_index(cs)
                    csrc = _lane_gather(sr, inv)
                    cdl = _lane_gather(d - lo, inv)
                    t = lc + pc
                    emit = t >= 16
                    sh = jnp.maximum(iota - lc, 0)
                    m_src = jnp.where(iota < lc, lv_src,
                                      _lane_gather(csrc, sh))
                    m_dl = jnp.where(iota < lc, lv_dl,
                                     _lane_gather(cdl, sh))
                    shh = jnp.minimum(iota + 16 - lc, 15)
                    h_src = _lane_gather(csrc, shh)
                    h_dl = _lane_gather(cdl, shh)

                    @pl.when(emit)
                    def _():
                        qsrc[pl.ds(qn, 16)] = m_src
                        qdl[pl.ds(qn, 16)] = m_dl

                    qn = jnp.where(emit, qn + 16, qn)
                    lc = t - jnp.where(emit, jnp.int32(16), jnp.int32(0))
                    lv_src = jnp.where(emit, h_src, m_src)
                    lv_dl = jnp.where(emit, h_dl, m_dl)
                fire = qn >= _CH
                for pp in range(2):
                    @pl.when(fire & ((fc % 2) == pp))
                    def _():
                        _fire(pp, fc)
                return (jnp.where(fire, qn - _CH, qn), lc,
                        fc + jnp.where(fire, jnp.int32(1), jnp.int32(0)),
                        lv_src, lv_dl)

            _issue_idx(0, 0)

            def pair(p, st_):
                st_ = _chunk(2 * p, 0, *st_)
                st_ = _chunk(2 * p + 1, 1, *st_)
                return st_

            qn, lc, fc, lv_src, lv_dl = lax.fori_loop(
                0, nk // 2, pair,
                (jnp.int32(0), jnp.int32(0), jnp.int32(0), _zero16,
                 _zero16))
            qsrc[pl.ds(qn, 16)] = lv_src
            qdl[pl.ds(qn, 16)] = lv_dl
            qtot = qn + lc
            for pp in range(2):
                @pl.when((fc % 2) == pp)
                def _():
                    _drain(pp, fc, qtot)

        groups = (
            (((x1, s101, d101),), out_a),
            (((x0, s021, d021),), out_b),
            (((x1, s110, d110), (x0, s030, d030)), out_c),
        )
        for arrays, out in groups:
            def _zrow(r, _):
                for j in range(_D // 16):
                    rows_v[0][r, pl.ds(j * 16, 16)] = jnp.zeros((16,), f32)
                return 0
            lax.fori_loop(0, _CH, _zrow, 0)
            nfull = q // _CH
            for k in range(nfull):
                pltpu.sync_copy(rows_v[0], acc.at[pl.ds(base + k * _CH, _CH)])
            rem = q - nfull * _CH
            if rem:
                pltpu.sync_copy(rows_v[0].at[pl.ds(0, rem)],
                                acc.at[pl.ds(base + nfull * _CH, rem)])
            plsc.subcore_barrier()
            for xt, st, dt in arrays:
                _scan_edges(xt, st, dt)
            plsc.subcore_barrier()
            ragged = (c == _NC - 1) & (s == _NT - 1)

            @pl.when(jnp.logical_not(ragged))
            def _():
                pltpu.sync_copy(acc.at[pl.ds(base, q)],
                                out.at[pl.ds(lo + base, q)])

            @pl.when(ragged)
            def _():
                pltpu.sync_copy(acc.at[pl.ds(base, last)],
                                out.at[pl.ds(lo + base, last)])

            plsc.subcore_barrier()

    return seg


_R = 1000


def _full(i):
    return (0, 0)


def _rowblk(i):
    return (i, 0)


@functools.lru_cache(maxsize=None)
def _build_tc_type1(n):
    grid = -(-n // _R)

    def body(x1, a, b_, gw1, gb1, gw2, gb2, hw, hb, out, stats):
        i = pl.program_id(0)
        gin = x1[...] + a[...]
        t = jnp.maximum(gin @ gw1[...] + gb1[...], 0.0) @ gw2[...] + gb2[...]
        h = (t + (b_[...] @ hw[...] + hb[...]) * _COEF) * 0.5
        hr = jnp.maximum(h, 0.0)
        out[...] = hr

        @pl.when(i == 0)
        def _():
            stats[...] = jnp.zeros_like(stats)

        stats[0:1, :] += jnp.sum(hr, axis=0, keepdims=True)
        stats[1:2, :] += jnp.sum(hr * hr, axis=0, keepdims=True)

    blk = pl.BlockSpec((_R, _D), _rowblk)
    wblk = pl.BlockSpec((_D, _D), _full)
    bblk = pl.BlockSpec((1, _D), _full)
    return pl.pallas_call(
        body,
        grid=(grid,),
        in_specs=[blk, blk, blk, wblk, bblk, wblk, bblk, wblk, bblk],
        out_specs=[pl.BlockSpec((_R, _D), _rowblk),
                   pl.BlockSpec((8, _D), _full)],
        out_shape=[jax.ShapeDtypeStruct((n, _D), jnp.float32),
                   jax.ShapeDtypeStruct((8, _D), jnp.float32)],
    )


@functools.lru_cache(maxsize=None)
def _build_tc_type0(n):
    grid = -(-n // _R)

    def body(cacc, hw, hb, out, stats):
        i = pl.program_id(0)
        h = (cacc[...] @ hw[...]) * (0.5 * _COEF) + hb[...] * _COEF
        hr = jnp.maximum(h, 0.0)
        out[...] = hr

        @pl.when(i == 0)
        def _():
            stats[...] = jnp.zeros_like(stats)

        stats[0:1, :] += jnp.sum(hr, axis=0, keepdims=True)
        stats[1:2, :] += jnp.sum(hr * hr, axis=0, keepdims=True)

    blk = pl.BlockSpec((_R, _D), _rowblk)
    return pl.pallas_call(
        body,
        grid=(grid,),
        in_specs=[blk, pl.BlockSpec((_D, _D), _full),
                  pl.BlockSpec((1, _D), _full)],
        out_specs=[pl.BlockSpec((_R, _D), _rowblk),
                   pl.BlockSpec((8, _D), _full)],
        out_shape=[jax.ShapeDtypeStruct((n, _D), jnp.float32),
                   jax.ShapeDtypeStruct((8, _D), jnp.float32)],
    )


@functools.lru_cache(maxsize=None)
def _build_tc_norm(n):
    grid = -(-n // _R)
    inv_n = 1.0 / n

    def body(hr, stats, g, b, out):
        st = stats[...]
        m = st[0:1] * inv_n
        v = st[1:2] * inv_n - m * m
        scale = g[...] * lax.rsqrt(v + _BN_EPS)
        out[...] = hr[...] * scale + (b[...] - m * scale)

    blk = pl.BlockSpec((_R, _D), _rowblk)
    return pl.pallas_call(
        body,
        grid=(grid,),
        in_specs=[blk, pl.BlockSpec((8, _D), _full),
                  pl.BlockSpec((1, _D), _full), pl.BlockSpec((1, _D), _full)],
        out_specs=blk,
        out_shape=jax.ShapeDtypeStruct((n, _D), jnp.float32),
    )


_EPAD = -(-_E // (_CH * 2 * _NT)) * (_CH * 2 * _NT)


def _layer(h0, h1, edges, gw1, gb1, gw2, gb2, hw, hb, bng, bnb):
    seg = _build_sc_segsum(_N, _EPAD)
    a, b_, cacc = seg(h0, h1, *edges)
    r2 = lambda v: v.reshape(1, _D)
    h1r, st1 = _build_tc_type1(_N)(h1, a, b_, gw1, r2(gb1), gw2, r2(gb2),
                                   hw, r2(hb))
    h0r, st0 = _build_tc_type0(_N)(cacc, hw, r2(hb))
    norm = _build_tc_norm(_N)
    h0n = norm(h0r, st0, r2(bng), r2(bnb))
    h1n = norm(h1r, st1, r2(bng), r2(bnb))
    return h0n, h1n


def kernel(x0, x1, ei_101, ei_110, ei_021, ei_030,
           gin0_w1, gin0_b1, gin0_w2, gin0_b2, hl0_w, hl0_b, bn0_g, bn0_b,
           gin1_w1, gin1_b1, gin1_w2, gin1_b2, hl1_w, hl1_b, bn1_g, bn1_b):
    i32 = jnp.int32
    spad = jnp.zeros((_EPAD - _E,), i32)
    dpad = jnp.full((_EPAD - _E,), _N, i32)

    def _src(a):
        return jnp.concatenate([a[0].astype(i32), spad])

    def _dst(a):
        return jnp.concatenate([a[1].astype(i32), dpad])

    edges = (_src(ei_101), _dst(ei_101), _src(ei_021), _dst(ei_021),
             _src(ei_110), _dst(ei_110), _src(ei_030), _dst(ei_030))
    h0, h1 = _layer(x0, x1, edges,
                    gin0_w1, gin0_b1, gin0_w2, gin0_b2, hl0_w, hl0_b,
                    bn0_g, bn0_b)
    h0, h1 = _layer(h0, h1, edges,
                    gin1_w1, gin1_b1, gin1_w2, gin1_b2, hl1_w, hl1_b,
                    bn1_g, bn1_b)
    return jnp.concatenate([h0, h1], axis=0)


# speedup vs baseline: 1.1243x; 1.1243x over previous
import functools

import jax
import jax.numpy as jnp
from jax import lax
from jax.experimental import pallas as pl
from jax.experimental.pallas import tpu as pltpu
from jax.experimental.pallas import tpu_sc as plsc

_N = 25000
_E = 150000
_D = 128
_COEF = 0.1
_BN_EPS = 1e-5

_NC = 2
_NT = 16
_CH = 112


@functools.lru_cache(maxsize=None)
def _build_sc_segsum(n, e):
    nch = -(-e // _CH)
    q = ((n + _NC * _NT - 1) // (_NC * _NT) + 7) // 8 * 8
    split = _NT * q
    trash = split
    acc_rows = split + 4 * _NT
    last = n - split - (_NT - 1) * q
    assert 0 < last <= q and split <= n and e % 8 == 0

    mesh = plsc.VectorSubcoreMesh(core_axis_name="c", subcore_axis_name="s")
    f32 = jnp.float32
    osd = jax.ShapeDtypeStruct((n, _D), f32)

    @functools.partial(
        pl.kernel,
        out_type=(osd, osd, osd),
        mesh=mesh,
        scratch_types=[
            pltpu.VMEM_SHARED((acc_rows, _D), f32),
            [pltpu.VMEM((_CH,), jnp.int32)] * 2,
            [pltpu.VMEM((_CH,), jnp.int32)] * 2,
            [pltpu.VMEM((_CH,), jnp.int32)] * 2,
            [pltpu.VMEM((_CH, _D), f32)] * 2,
            pltpu.SemaphoreType.DMA,
            pltpu.SemaphoreType.DMA,
            pltpu.SemaphoreType.DMA,
        ],
    )
    def seg(x0, x1, s101, d101, s021, d021, s110, d110, s030, d030,
            out_a, out_b, out_c, acc, src_v, dst_v, dl_v, rows_v,
            sem_i, sem_g, sem_s):
        c = lax.axis_index("c")
        s = lax.axis_index("s")
        lo = c * split
        hi = jnp.where(c == 0, split, n)
        base = s * q

        def _scan_edges(xt, st, dt):
            nk = (nch - 1 - s) // _NT + 1

            def _off(k):
                start = (s + k * _NT) * _CH
                return start, jnp.minimum(start, e - _CH)

            def _issue_idx(k, b):
                _, off = _off(k)
                pltpu.async_copy(st.at[pl.ds(off, _CH)], src_v[b], sem_i)
                pltpu.async_copy(dt.at[pl.ds(off, _CH)], dst_v[b], sem_i)

            def _wait_idx(k, b):
                _, off = _off(k)
                pltpu.make_async_copy(st.at[pl.ds(off, _CH)], src_v[b],
                                      sem_i).wait()
                pltpu.make_async_copy(dt.at[pl.ds(off, _CH)], dst_v[b],
                                      sem_i).wait()

            tr = trash + s * 4 + (lax.iota(jnp.int32, 16) & 3)

            def _chunk(k, b):
                @pl.when(k > 0)
                def _():
                    pltpu.make_async_copy(xt.at[src_v[1 - b]],
                                          rows_v[1 - b], sem_g).wait()

                @pl.when(k > 1)
                def _():
                    pltpu.make_async_copy(rows_v[b], acc.at[dl_v[b]],
                                          sem_s).wait()

                @pl.when(k + 1 < nk)
                def _():
                    _issue_idx(k + 1, 1 - b)

                _wait_idx(k, b)
                start, off = _off(k)
                for j in range(_CH // 16):
                    d = dst_v[b][pl.ds(j * 16, 16)]
                    eid = off + j * 16 + lax.iota(jnp.int32, 16)
                    ok = (eid >= start) & (d >= lo) & (d < hi)
                    dl_v[b][pl.ds(j * 16, 16)] = jnp.where(ok, d - lo, tr)

                pltpu.async_copy(xt.at[src_v[b]], rows_v[b], sem_g)

                @pl.when(k > 0)
                def _():
                    pltpu.async_copy(rows_v[1 - b], acc.at[dl_v[1 - b]],
                                     sem_s, add=True)

            _issue_idx(0, 0)

            def body(p, _):
                _chunk(2 * p, 0)
                k = 2 * p + 1

                @pl.when(k < nk)
                def _():
                    _chunk(k, 1)

                return 0

            lax.fori_loop(0, (nk + 1) // 2, body, 0)

            for b in range(2):
                @pl.when((nk - 1) % 2 == b)
                def _():
                    pltpu.make_async_copy(xt.at[src_v[b]], rows_v[b],
                                          sem_g).wait()
                    pltpu.async_copy(rows_v[b], acc.at[dl_v[b]], sem_s,
                                     add=True)
                    pltpu.make_async_copy(rows_v[1 - b],
                                          acc.at[dl_v[1 - b]], sem_s).wait()
                    pltpu.make_async_copy(rows_v[b], acc.at[dl_v[b]],
                                          sem_s).wait()

        groups = (
            (((x1, s101, d101),), out_a),
            (((x0, s021, d021),), out_b),
            (((x1, s110, d110), (x0, s030, d030)), out_c),
        )
        for arrays, out in groups:
            def _zrow(r, _):
                for j in range(_D // 16):
                    rows_v[0][r, pl.ds(j * 16, 16)] = jnp.zeros((16,), f32)
                return 0
            lax.fori_loop(0, _CH, _zrow, 0)
            nfull = q // _CH
            for k in range(nfull):
                pltpu.sync_copy(rows_v[0], acc.at[pl.ds(base + k * _CH, _CH)])
            rem = q - nfull * _CH
            if rem:
                pltpu.sync_copy(rows_v[0].at[pl.ds(0, rem)],
                                acc.at[pl.ds(base + nfull * _CH, rem)])
            plsc.subcore_barrier()
            for xt, st, dt in arrays:
                _scan_edges(xt, st, dt)
            plsc.subcore_barrier()
            ragged = (c == _NC - 1) & (s == _NT - 1)

            @pl.when(jnp.logical_not(ragged))
            def _():
                pltpu.sync_copy(acc.at[pl.ds(base, q)],
                                out.at[pl.ds(lo + base, q)])

            @pl.when(ragged)
            def _():
                pltpu.sync_copy(acc.at[pl.ds(base, last)],
                                out.at[pl.ds(lo + base, last)])

            plsc.subcore_barrier()

    return seg


_R = 1000


def _full(i):
    return (0, 0)


def _rowblk(i):
    return (i, 0)


@functools.lru_cache(maxsize=None)
def _build_tc_compute(n):
    grid = -(-n // _R)

    def body(x1, a, b_, cacc, gw1, gb1, gw2, gb2, hw, hb,
             out1, out0, st1, st0):
        i = pl.program_id(0)
        hwv = hw[...]
        hbv = hb[...]
        gin = x1[...] + a[...]
        t = jnp.maximum(gin @ gw1[...] + gb1[...], 0.0) @ gw2[...] + gb2[...]
        h1 = (t + (b_[...] @ hwv + hbv) * _COEF) * 0.5
        h1r = jnp.maximum(h1, 0.0)
        h0 = (cacc[...] @ hwv) * (0.5 * _COEF) + hbv * _COEF
        h0r = jnp.maximum(h0, 0.0)
        out1[...] = h1r
        out0[...] = h0r

        @pl.when(i == 0)
        def _():
            st1[...] = jnp.zeros_like(st1)
            st0[...] = jnp.zeros_like(st0)

        st1[0:1, :] += jnp.sum(h1r, axis=0, keepdims=True)
        st1[1:2, :] += jnp.sum(h1r * h1r, axis=0, keepdims=True)
        st0[0:1, :] += jnp.sum(h0r, axis=0, keepdims=True)
        st0[1:2, :] += jnp.sum(h0r * h0r, axis=0, keepdims=True)

    blk = pl.BlockSpec((_R, _D), _rowblk)
    wblk = pl.BlockSpec((_D, _D), _full)
    bblk = pl.BlockSpec((1, _D), _full)
    sblk = pl.BlockSpec((8, _D), _full)
    ssd = jax.ShapeDtypeStruct((8, _D), jnp.float32)
    osd = jax.ShapeDtypeStruct((n, _D), jnp.float32)
    return pl.pallas_call(
        body,
        grid=(grid,),
        in_specs=[blk, blk, blk, blk, wblk, bblk, wblk, bblk, wblk, bblk],
        out_specs=[blk, blk, sblk, sblk],
        out_shape=[osd, osd, ssd, ssd],
    )


def _bn_apply(hr_blk, st, g, b, inv_n):
    m = st[0:1] * inv_n
    v = st[1:2] * inv_n - m * m
    scale = g * lax.rsqrt(v + _BN_EPS)
    return hr_blk * scale + (b - m * scale)


@functools.lru_cache(maxsize=None)
def _build_tc_norm2(n):
    grid = -(-n // _R)
    inv_n = 1.0 / n

    def body(h0r, h1r, st0, st1, g, b, o0, o1):
        gv, bv = g[...], b[...]
        o0[...] = _bn_apply(h0r[...], st0[...], gv, bv, inv_n)
        o1[...] = _bn_apply(h1r[...], st1[...], gv, bv, inv_n)

    blk = pl.BlockSpec((_R, _D), _rowblk)
    sblk = pl.BlockSpec((8, _D), _full)
    bblk = pl.BlockSpec((1, _D), _full)
    osd = jax.ShapeDtypeStruct((n, _D), jnp.float32)
    return pl.pallas_call(
        body,
        grid=(grid,),
        in_specs=[blk, blk, sblk, sblk, bblk, bblk],
        out_specs=[blk, blk],
        out_shape=[osd, osd],
    )


@functools.lru_cache(maxsize=None)
def _build_tc_norm_cat(n):
    nb = -(-n // _R)
    grid = 2 * nb
    inv_n = 1.0 / n

    def body(h0r, h1r, st0, st1, g, b, out):
        i = pl.program_id(0)
        gv, bv = g[...], b[...]
        y0 = _bn_apply(h0r[...], st0[...], gv, bv, inv_n)
        y1 = _bn_apply(h1r[...], st1[...], gv, bv, inv_n)
        out[...] = jnp.where(i < nb, y0, y1)

    blk0 = pl.BlockSpec((_R, _D), lambda i: (jnp.minimum(i, nb - 1), 0))
    blk1 = pl.BlockSpec((_R, _D), lambda i: (jnp.maximum(i - nb, 0), 0))
    sblk = pl.BlockSpec((8, _D), _full)
    bblk = pl.BlockSpec((1, _D), _full)
    return pl.pallas_call(
        body,
        grid=(grid,),
        in_specs=[blk0, blk1, sblk, sblk, bblk, bblk],
        out_specs=pl.BlockSpec((_R, _D), _rowblk),
        out_shape=jax.ShapeDtypeStruct((2 * n, _D), jnp.float32),
    )


def _layer(h0, h1, edges, gw1, gb1, gw2, gb2, hw, hb, bng, bnb, final):
    seg = _build_sc_segsum(_N, _E)
    a, b_, cacc = seg(h0, h1, *edges)
    r2 = lambda v: v.reshape(1, _D)
    h1r, h0r, st1, st0 = _build_tc_compute(_N)(
        h1, a, b_, cacc, gw1, r2(gb1), gw2, r2(gb2), hw, r2(hb))
    if final:
        return _build_tc_norm_cat(_N)(h0r, h1r, st0, st1, r2(bng), r2(bnb))
    return _build_tc_norm2(_N)(h0r, h1r, st0, st1, r2(bng), r2(bnb))


def kernel(x0, x1, ei_101, ei_110, ei_021, ei_030,
           gin0_w1, gin0_b1, gin0_w2, gin0_b2, hl0_w, hl0_b, bn0_g, bn0_b,
           gin1_w1, gin1_b1, gin1_w2, gin1_b2, hl1_w, hl1_b, bn1_g, bn1_b):
    i32 = jnp.int32
    edges = (ei_101[0].astype(i32), ei_101[1].astype(i32),
             ei_021[0].astype(i32), ei_021[1].astype(i32),
             ei_110[0].astype(i32), ei_110[1].astype(i32),
             ei_030[0].astype(i32), ei_030[1].astype(i32))
    h0, h1 = _layer(x0, x1, edges,
                    gin0_w1, gin0_b1, gin0_w2, gin0_b2, hl0_w, hl0_b,
                    bn0_g, bn0_b, final=False)
    return _layer(h0, h1, edges,
                  gin1_w1, gin1_b1, gin1_w2, gin1_b2, hl1_w, hl1_b,
                  bn1_g, bn1_b, final=True)


# speedup vs baseline: 1.1452x; 1.0186x over previous
import functools

import jax
import jax.numpy as jnp
from jax import lax
from jax.experimental import pallas as pl
from jax.experimental.pallas import tpu as pltpu
from jax.experimental.pallas import tpu_sc as plsc

_N = 25000
_E = 150000
_D = 128
_COEF = 0.1
_BN_EPS = 1e-5

_NC = 2
_NT = 16
_CH = 112


@functools.lru_cache(maxsize=None)
def _build_sc_segsum(n, e, kind):
    nch = -(-e // _CH)
    q = ((n + _NC * _NT - 1) // (_NC * _NT) + 7) // 8 * 8
    split = _NT * q
    trash = split
    acc_rows = split + 4 * _NT
    last = n - split - (_NT - 1) * q
    assert 0 < last <= q and split <= n and e % 8 == 0

    mesh = plsc.VectorSubcoreMesh(core_axis_name="c", subcore_axis_name="s")
    f32 = jnp.float32
    osd = jax.ShapeDtypeStruct((n, _D), f32)
    nout = 2 if kind == "ab" else 1

    @functools.partial(
        pl.kernel,
        out_type=(osd,) * nout,
        mesh=mesh,
        scratch_types=[
            pltpu.VMEM_SHARED((acc_rows, _D), f32),
            [pltpu.VMEM((_CH,), jnp.int32)] * 2,
            [pltpu.VMEM((_CH,), jnp.int32)] * 2,
            [pltpu.VMEM((_CH,), jnp.int32)] * 2,
            [pltpu.VMEM((_CH, _D), f32)] * 2,
            pltpu.SemaphoreType.DMA,
            pltpu.SemaphoreType.DMA,
            pltpu.SemaphoreType.DMA,
        ],
    )
    def seg(*refs):
        x0, x1, s1, d1, s2, d2 = refs[:6]
        outs = refs[6:6 + nout]
        (acc, src_v, dst_v, dl_v, rows_v,
         sem_i, sem_g, sem_s) = refs[6 + nout:]
        c = lax.axis_index("c")
        s = lax.axis_index("s")
        lo = c * split
        hi = jnp.where(c == 0, split, n)
        base = s * q

        def _scan_edges(xt, st, dt):
            nk = (nch - 1 - s) // _NT + 1

            def _off(k):
                start = (s + k * _NT) * _CH
                return start, jnp.minimum(start, e - _CH)

            def _issue_idx(k, b):
                _, off = _off(k)
                pltpu.async_copy(st.at[pl.ds(off, _CH)], src_v[b], sem_i)
                pltpu.async_copy(dt.at[pl.ds(off, _CH)], dst_v[b], sem_i)

            def _wait_idx(k, b):
                _, off = _off(k)
                pltpu.make_async_copy(st.at[pl.ds(off, _CH)], src_v[b],
                                      sem_i).wait()
                pltpu.make_async_copy(dt.at[pl.ds(off, _CH)], dst_v[b],
                                      sem_i).wait()

            tr = trash + s * 4 + (lax.iota(jnp.int32, 16) & 3)

            def _chunk(k, b):
                @pl.when(k > 0)
                def _():
                    pltpu.make_async_copy(xt.at[src_v[1 - b]],
                                          rows_v[1 - b], sem_g).wait()

                @pl.when(k > 1)
                def _():
                    pltpu.make_async_copy(rows_v[b], acc.at[dl_v[b]],
                                          sem_s).wait()

                @pl.when(k + 1 < nk)
                def _():
                    _issue_idx(k + 1, 1 - b)

                _wait_idx(k, b)
                start, off = _off(k)
                for j in range(_CH // 16):
                    d = dst_v[b][pl.ds(j * 16, 16)]
                    eid = off + j * 16 + lax.iota(jnp.int32, 16)
                    ok = (eid >= start) & (d >= lo) & (d < hi)
                    dl_v[b][pl.ds(j * 16, 16)] = jnp.where(ok, d - lo, tr)

                pltpu.async_copy(xt.at[src_v[b]], rows_v[b], sem_g)

                @pl.when(k > 0)
                def _():
                    pltpu.async_copy(rows_v[1 - b], acc.at[dl_v[1 - b]],
                                     sem_s, add=True)

            _issue_idx(0, 0)

            def body(p, _):
                _chunk(2 * p, 0)
                k = 2 * p + 1

                @pl.when(k < nk)
                def _():
                    _chunk(k, 1)

                return 0

            lax.fori_loop(0, (nk + 1) // 2, body, 0)

            for b in range(2):
                @pl.when((nk - 1) % 2 == b)
                def _():
                    pltpu.make_async_copy(xt.at[src_v[b]], rows_v[b],
                                          sem_g).wait()
                    pltpu.async_copy(rows_v[b], acc.at[dl_v[b]], sem_s,
                                     add=True)
                    pltpu.make_async_copy(rows_v[1 - b],
                                          acc.at[dl_v[1 - b]], sem_s).wait()
                    pltpu.make_async_copy(rows_v[b], acc.at[dl_v[b]],
                                          sem_s).wait()

        if kind == "ab":
            groups = ((((x1, s1, d1),), outs[0]),
                      (((x0, s2, d2),), outs[1]))
        else:
            groups = ((((x1, s1, d1), (x0, s2, d2)), outs[0]),)
        for arrays, out in groups:
            def _zrow(r, _):
                for j in range(_D // 16):
                    rows_v[0][r, pl.ds(j * 16, 16)] = jnp.zeros((16,), f32)
                return 0
            lax.fori_loop(0, _CH, _zrow, 0)
            nfull = q // _CH
            for k in range(nfull):
                pltpu.sync_copy(rows_v[0], acc.at[pl.ds(base + k * _CH, _CH)])
            rem = q - nfull * _CH
            if rem:
                pltpu.sync_copy(rows_v[0].at[pl.ds(0, rem)],
                                acc.at[pl.ds(base + nfull * _CH, rem)])
            plsc.subcore_barrier()
            for xt, st, dt in arrays:
                _scan_edges(xt, st, dt)
            plsc.subcore_barrier()
            ragged = (c == _NC - 1) & (s == _NT - 1)

            @pl.when(jnp.logical_not(ragged))
            def _():
                pltpu.sync_copy(acc.at[pl.ds(base, q)],
                                out.at[pl.ds(lo + base, q)])

            @pl.when(ragged)
            def _():
                pltpu.sync_copy(acc.at[pl.ds(base, last)],
                                out.at[pl.ds(lo + base, last)])

            plsc.subcore_barrier()

    return seg


_R = 1000


def _full(i):
    return (0, 0)


def _rowblk(i):
    return (i, 0)


@functools.lru_cache(maxsize=None)
def _build_tc_type1(n):
    grid = -(-n // _R)

    def body(x1, a, b_, gw1, gb1, gw2, gb2, hw, hb, out1, st1):
        i = pl.program_id(0)
        gin = x1[...] + a[...]
        t = jnp.maximum(gin @ gw1[...] + gb1[...], 0.0) @ gw2[...] + gb2[...]
        h1 = (t + (b_[...] @ hw[...] + hb[...]) * _COEF) * 0.5
        h1r = jnp.maximum(h1, 0.0)
        out1[...] = h1r

        @pl.when(i == 0)
        def _():
            st1[...] = jnp.zeros_like(st1)

        st1[0:1, :] += jnp.sum(h1r, axis=0, keepdims=True)
        st1[1:2, :] += jnp.sum(h1r * h1r, axis=0, keepdims=True)

    blk = pl.BlockSpec((_R, _D), _rowblk)
    wblk = pl.BlockSpec((_D, _D), _full)
    bblk = pl.BlockSpec((1, _D), _full)
    sblk = pl.BlockSpec((8, _D), _full)
    return pl.pallas_call(
        body,
        grid=(grid,),
        in_specs=[blk, blk, blk, wblk, bblk, wblk, bblk, wblk, bblk],
        out_specs=[blk, sblk],
        out_shape=[jax.ShapeDtypeStruct((n, _D), jnp.float32),
                   jax.ShapeDtypeStruct((8, _D), jnp.float32)],
    )


@functools.lru_cache(maxsize=None)
def _build_tc_type0(n):
    grid = -(-n // _R)

    def body(cacc, hw, hb, out0, st0):
        i = pl.program_id(0)
        h0 = (cacc[...] @ hw[...]) * (0.5 * _COEF) + hb[...] * _COEF
        h0r = jnp.maximum(h0, 0.0)
        out0[...] = h0r

        @pl.when(i == 0)
        def _():
            st0[...] = jnp.zeros_like(st0)

        st0[0:1, :] += jnp.sum(h0r, axis=0, keepdims=True)
        st0[1:2, :] += jnp.sum(h0r * h0r, axis=0, keepdims=True)

    blk = pl.BlockSpec((_R, _D), _rowblk)
    return pl.pallas_call(
        body,
        grid=(grid,),
        in_specs=[blk, pl.BlockSpec((_D, _D), _full),
                  pl.BlockSpec((1, _D), _full)],
        out_specs=[blk, pl.BlockSpec((8, _D), _full)],
        out_shape=[jax.ShapeDtypeStruct((n, _D), jnp.float32),
                   jax.ShapeDtypeStruct((8, _D), jnp.float32)],
    )


def _bn_apply(hr_blk, st, g, b, inv_n):
    m = st[0:1] * inv_n
    v = st[1:2] * inv_n - m * m
    scale = g * lax.rsqrt(v + _BN_EPS)
    return hr_blk * scale + (b - m * scale)


@functools.lru_cache(maxsize=None)
def _build_tc_norm2(n):
    grid = -(-n // _R)
    inv_n = 1.0 / n

    def body(h0r, h1r, st0, st1, g, b, o0, o1):
        gv, bv = g[...], b[...]
        o0[...] = _bn_apply(h0r[...], st0[...], gv, bv, inv_n)
        o1[...] = _bn_apply(h1r[...], st1[...], gv, bv, inv_n)

    blk = pl.BlockSpec((_R, _D), _rowblk)
    sblk = pl.BlockSpec((8, _D), _full)
    bblk = pl.BlockSpec((1, _D), _full)
    osd = jax.ShapeDtypeStruct((n, _D), jnp.float32)
    return pl.pallas_call(
        body,
        grid=(grid,),
        in_specs=[blk, blk, sblk, sblk, bblk, bblk],
        out_specs=[blk, blk],
        out_shape=[osd, osd],
    )


@functools.lru_cache(maxsize=None)
def _build_tc_norm_cat(n):
    nb = -(-n // _R)
    grid = 2 * nb
    inv_n = 1.0 / n

    def body(h0r, h1r, st0, st1, g, b, out):
        i = pl.program_id(0)
        gv, bv = g[...], b[...]
        y0 = _bn_apply(h0r[...], st0[...], gv, bv, inv_n)
        y1 = _bn_apply(h1r[...], st1[...], gv, bv, inv_n)
        out[...] = jnp.where(i < nb, y0, y1)

    blk0 = pl.BlockSpec((_R, _D), lambda i: (jnp.minimum(i, nb - 1), 0))
    blk1 = pl.BlockSpec((_R, _D), lambda i: (jnp.maximum(i - nb, 0), 0))
    sblk = pl.BlockSpec((8, _D), _full)
    bblk = pl.BlockSpec((1, _D), _full)
    return pl.pallas_call(
        body,
        grid=(grid,),
        in_specs=[blk0, blk1, sblk, sblk, bblk, bblk],
        out_specs=pl.BlockSpec((_R, _D), _rowblk),
        out_shape=jax.ShapeDtypeStruct((2 * n, _D), jnp.float32),
    )


def _layer(h0, h1, edges, gw1, gb1, gw2, gb2, hw, hb, bng, bnb, final):
    s101, d101, s021, d021, s110, d110, s030, d030 = edges
    a, b_ = _build_sc_segsum(_N, _E, "ab")(h0, h1, s101, d101, s021, d021)
    (cacc,) = _build_sc_segsum(_N, _E, "c")(h0, h1, s110, d110, s030, d030)
    r2 = lambda v: v.reshape(1, _D)
    h1r, st1 = _build_tc_type1(_N)(h1, a, b_, gw1, r2(gb1), gw2, r2(gb2),
                                   hw, r2(hb))
    h0r, st0 = _build_tc_type0(_N)(cacc, hw, r2(hb))
    if final:
        return _build_tc_norm_cat(_N)(h0r, h1r, st0, st1, r2(bng), r2(bnb))
    return _build_tc_norm2(_N)(h0r, h1r, st0, st1, r2(bng), r2(bnb))


def kernel(x0, x1, ei_101, ei_110, ei_021, ei_030,
           gin0_w1, gin0_b1, gin0_w2, gin0_b2, hl0_w, hl0_b, bn0_g, bn0_b,
           gin1_w1, gin1_b1, gin1_w2, gin1_b2, hl1_w, hl1_b, bn1_g, bn1_b):
    i32 = jnp.int32
    edges = (ei_101[0].astype(i32), ei_101[1].astype(i32),
             ei_021[0].astype(i32), ei_021[1].astype(i32),
             ei_110[0].astype(i32), ei_110[1].astype(i32),
             ei_030[0].astype(i32), ei_030[1].astype(i32))
    h0, h1 = _layer(x0, x1, edges,
                    gin0_w1, gin0_b1, gin0_w2, gin0_b2, hl0_w, hl0_b,
                    bn0_g, bn0_b, final=False)
    return _layer(h0, h1, edges,
                  gin1_w1, gin1_b1, gin1_w2, gin1_b2, hl1_w, hl1_b,
                  bn1_g, bn1_b, final=True)


# speedup vs baseline: 1.1503x; 1.0044x over previous
import functools

import jax
import jax.numpy as jnp
from jax import lax
from jax.experimental import pallas as pl
from jax.experimental.pallas import tpu as pltpu
from jax.experimental.pallas import tpu_sc as plsc

_N = 25000
_E = 150000
_D = 128
_COEF = 0.1
_BN_EPS = 1e-5

_NC = 2
_NT = 16
_CH = 112


@functools.lru_cache(maxsize=None)
def _build_sc_segsum(n, e, kind):
    nch = -(-e // _CH)
    q = ((n + _NC * _NT - 1) // (_NC * _NT) + 7) // 8 * 8
    split = _NT * q
    trash = split
    acc_rows = split + 4 * _NT
    last = n - split - (_NT - 1) * q
    assert 0 < last <= q and split <= n and e % 8 == 0

    mesh = plsc.VectorSubcoreMesh(core_axis_name="c", subcore_axis_name="s")
    f32 = jnp.float32
    osd = jax.ShapeDtypeStruct((n, _D), f32)
    nout = 2 if kind == "ab" else 1

    @functools.partial(
        pl.kernel,
        out_type=(osd,) * nout,
        mesh=mesh,
        scratch_types=[
            pltpu.VMEM_SHARED((acc_rows, _D), f32),
            [pltpu.VMEM((_CH,), jnp.int32)] * 2,
            [pltpu.VMEM((_CH,), jnp.int32)] * 2,
            [pltpu.VMEM((_CH,), jnp.int32)] * 2,
            [pltpu.VMEM((_CH, _D), f32)] * 2,
            pltpu.SemaphoreType.DMA,
            pltpu.SemaphoreType.DMA,
            pltpu.SemaphoreType.DMA,
        ],
    )
    def seg(*refs):
        x0, x1, s1, d1, s2, d2 = refs[:6]
        outs = refs[6:6 + nout]
        (acc, src_v, dst_v, dl_v, rows_v,
         sem_i, sem_g, sem_s) = refs[6 + nout:]
        c = lax.axis_index("c")
        s = lax.axis_index("s")
        lo = c * split
        hi = jnp.where(c == 0, split, n)
        base = s * q

        def _scan_edges(xt, st, dt):
            nk = (nch - 1 - s) // _NT + 1

            def _off(k):
                start = (s + k * _NT) * _CH
                return start, jnp.minimum(start, e - _CH)

            def _issue_idx(k, b):
                _, off = _off(k)
                pltpu.async_copy(st.at[pl.ds(off, _CH)], src_v[b], sem_i)
                pltpu.async_copy(dt.at[pl.ds(off, _CH)], dst_v[b], sem_i)

            def _wait_idx(k, b):
                _, off = _off(k)
                pltpu.make_async_copy(st.at[pl.ds(off, _CH)], src_v[b],
                                      sem_i).wait()
                pltpu.make_async_copy(dt.at[pl.ds(off, _CH)], dst_v[b],
                                      sem_i).wait()

            tr = trash + s * 4 + (lax.iota(jnp.int32, 16) & 3)

            def _chunk(k, b):
                @pl.when(k > 0)
                def _():
                    pltpu.make_async_copy(xt.at[src_v[1 - b]],
                                          rows_v[1 - b], sem_g).wait()

                @pl.when(k > 1)
                def _():
                    pltpu.make_async_copy(rows_v[b], acc.at[dl_v[b]],
                                          sem_s).wait()

                @pl.when(k + 1 < nk)
                def _():
                    _issue_idx(k + 1, 1 - b)

                _wait_idx(k, b)
                start, off = _off(k)
                for j in range(_CH // 16):
                    d = dst_v[b][pl.ds(j * 16, 16)]
                    eid = off + j * 16 + lax.iota(jnp.int32, 16)
                    ok = (eid >= start) & (d >= lo) & (d < hi)
                    dl_v[b][pl.ds(j * 16, 16)] = jnp.where(ok, d - lo, tr)

                pltpu.async_copy(xt.at[src_v[b]], rows_v[b], sem_g)

                @pl.when(k > 0)
                def _():
                    pltpu.async_copy(rows_v[1 - b], acc.at[dl_v[1 - b]],
                                     sem_s, add=True)

            _issue_idx(0, 0)

            def body(p, _):
                _chunk(2 * p, 0)
                k = 2 * p + 1

                @pl.when(k < nk)
                def _():
                    _chunk(k, 1)

                return 0

            lax.fori_loop(0, (nk + 1) // 2, body, 0)

            for b in range(2):
                @pl.when((nk - 1) % 2 == b)
                def _():
                    pltpu.make_async_copy(xt.at[src_v[b]], rows_v[b],
                                          sem_g).wait()
                    pltpu.async_copy(rows_v[b], acc.at[dl_v[b]], sem_s,
                                     add=True)
                    pltpu.make_async_copy(rows_v[1 - b],
                                          acc.at[dl_v[1 - b]], sem_s).wait()
                    pltpu.make_async_copy(rows_v[b], acc.at[dl_v[b]],
                                          sem_s).wait()

        if kind == "ab":
            groups = ((((x1, s1, d1),), outs[0]),
                      (((x0, s2, d2),), outs[1]))
        else:
            groups = ((((x1, s1, d1), (x0, s2, d2)), outs[0]),)
        for arrays, out in groups:
            def _zrow(r, _):
                for j in range(_D // 16):
                    rows_v[0][r, pl.ds(j * 16, 16)] = jnp.zeros((16,), f32)
                return 0
            lax.fori_loop(0, _CH, _zrow, 0)
            nfull = q // _CH
            for k in range(nfull):
                pltpu.sync_copy(rows_v[0], acc.at[pl.ds(base + k * _CH, _CH)])
            rem = q - nfull * _CH
            if rem:
                pltpu.sync_copy(rows_v[0].at[pl.ds(0, rem)],
                                acc.at[pl.ds(base + nfull * _CH, rem)])
            plsc.subcore_barrier()
            for xt, st, dt in arrays:
                _scan_edges(xt, st, dt)
            plsc.subcore_barrier()
            ragged = (c == _NC - 1) & (s == _NT - 1)

            @pl.when(jnp.logical_not(ragged))
            def _():
                pltpu.sync_copy(acc.at[pl.ds(base, q)],
                                out.at[pl.ds(lo + base, q)])

            @pl.when(ragged)
            def _():
                pltpu.sync_copy(acc.at[pl.ds(base, last)],
                                out.at[pl.ds(lo + base, last)])

            plsc.subcore_barrier()

    return seg


_R = 1000


def _full(i):
    return (0, 0)


def _rowblk(i):
    return (i, 0)


@functools.lru_cache(maxsize=None)
def _build_tc_type1(n):
    grid = -(-n // _R)

    def body(x1, a, b_, gw1, gb1, gw2, gb2, hw, hb, out1, st1):
        i = pl.program_id(0)
        gin = x1[...] + a[...]
        t = jnp.maximum(gin @ gw1[...] + gb1[...], 0.0) @ gw2[...] + gb2[...]
        h1 = (t + (b_[...] @ hw[...] + hb[...]) * _COEF) * 0.5
        h1r = jnp.maximum(h1, 0.0)
        out1[...] = h1r

        @pl.when(i == 0)
        def _():
            st1[...] = jnp.zeros_like(st1)

        st1[0:1, :] += jnp.sum(h1r, axis=0, keepdims=True)
        st1[1:2, :] += jnp.sum(h1r * h1r, axis=0, keepdims=True)

    blk = pl.BlockSpec((_R, _D), _rowblk)
    wblk = pl.BlockSpec((_D, _D), _full)
    bblk = pl.BlockSpec((1, _D), _full)
    sblk = pl.BlockSpec((8, _D), _full)
    return pl.pallas_call(
        body,
        grid=(grid,),
        in_specs=[blk, blk, blk, wblk, bblk, wblk, bblk, wblk, bblk],
        out_specs=[blk, sblk],
        out_shape=[jax.ShapeDtypeStruct((n, _D), jnp.float32),
                   jax.ShapeDtypeStruct((8, _D), jnp.float32)],
    )


@functools.lru_cache(maxsize=None)
def _build_tc_type0(n):
    grid = -(-n // _R)

    def body(cacc, hw, hb, out0, st0):
        i = pl.program_id(0)
        h0 = (cacc[...] @ hw[...]) * (0.5 * _COEF) + hb[...] * _COEF
        h0r = jnp.maximum(h0, 0.0)
        out0[...] = h0r

        @pl.when(i == 0)
        def _():
            st0[...] = jnp.zeros_like(st0)

        st0[0:1, :] += jnp.sum(h0r, axis=0, keepdims=True)
        st0[1:2, :] += jnp.sum(h0r * h0r, axis=0, keepdims=True)

    blk = pl.BlockSpec((_R, _D), _rowblk)
    return pl.pallas_call(
        body,
        grid=(grid,),
        in_specs=[blk, pl.BlockSpec((_D, _D), _full),
                  pl.BlockSpec((1, _D), _full)],
        out_specs=[blk, pl.BlockSpec((8, _D), _full)],
        out_shape=[jax.ShapeDtypeStruct((n, _D), jnp.float32),
                   jax.ShapeDtypeStruct((8, _D), jnp.float32)],
    )


def _bn_apply(hr_blk, st, g, b, inv_n):
    m = st[0:1] * inv_n
    v = st[1:2] * inv_n - m * m
    scale = g * lax.rsqrt(v + _BN_EPS)
    return hr_blk * scale + (b - m * scale)


@functools.lru_cache(maxsize=None)
def _build_tc_norm2(n):
    grid = -(-n // _R)
    inv_n = 1.0 / n

    def body(h0r, h1r, st0, st1, g, b, o0, o1):
        gv, bv = g[...], b[...]
        o0[...] = _bn_apply(h0r[...], st0[...], gv, bv, inv_n)
        o1[...] = _bn_apply(h1r[...], st1[...], gv, bv, inv_n)

    blk = pl.BlockSpec((_R, _D), _rowblk)
    sblk = pl.BlockSpec((8, _D), _full)
    bblk = pl.BlockSpec((1, _D), _full)
    osd = jax.ShapeDtypeStruct((n, _D), jnp.float32)
    return pl.pallas_call(
        body,
        grid=(grid,),
        in_specs=[blk, blk, sblk, sblk, bblk, bblk],
        out_specs=[blk, blk],
        out_shape=[osd, osd],
    )


@functools.lru_cache(maxsize=None)
def _build_tc_norm_cat(n):
    nb = -(-n // _R)
    grid = 2 * nb
    inv_n = 1.0 / n

    def body(h0r, h1r, st0, st1, g, b, out):
        i = pl.program_id(0)
        gv, bv = g[...], b[...]
        y0 = _bn_apply(h0r[...], st0[...], gv, bv, inv_n)
        y1 = _bn_apply(h1r[...], st1[...], gv, bv, inv_n)
        out[...] = jnp.where(i < nb, y0, y1)

    blk0 = pl.BlockSpec((_R, _D), lambda i: (jnp.minimum(i, nb - 1), 0))
    blk1 = pl.BlockSpec((_R, _D), lambda i: (jnp.maximum(i - nb, 0), 0))
    sblk = pl.BlockSpec((8, _D), _full)
    bblk = pl.BlockSpec((1, _D), _full)
    return pl.pallas_call(
        body,
        grid=(grid,),
        in_specs=[blk0, blk1, sblk, sblk, bblk, bblk],
        out_specs=pl.BlockSpec((_R, _D), _rowblk),
        out_shape=jax.ShapeDtypeStruct((2 * n, _D), jnp.float32),
    )


def _layer(h0, h1, edges, gw1, gb1, gw2, gb2, hw, hb, bng, bnb, final):
    s101, d101, s021, d021, s110, d110, s030, d030 = edges
    r2 = lambda v: v.reshape(1, _D)
    a, b_ = _build_sc_segsum(_N, _E, "ab")(h0, h1, s101, d101, s021, d021)
    h1r, st1 = _build_tc_type1(_N)(h1, a, b_, gw1, r2(gb1), gw2, r2(gb2),
                                   hw, r2(hb))
    (cacc,) = _build_sc_segsum(_N, _E, "c")(h0, h1, s110, d110, s030, d030)
    h0r, st0 = _build_tc_type0(_N)(cacc, hw, r2(hb))
    if final:
        return _build_tc_norm_cat(_N)(h0r, h1r, st0, st1, r2(bng), r2(bnb))
    return _build_tc_norm2(_N)(h0r, h1r, st0, st1, r2(bng), r2(bnb))


def kernel(x0, x1, ei_101, ei_110, ei_021, ei_030,
           gin0_w1, gin0_b1, gin0_w2, gin0_b2, hl0_w, hl0_b, bn0_g, bn0_b,
           gin1_w1, gin1_b1, gin1_w2, gin1_b2, hl1_w, hl1_b, bn1_g, bn1_b):
    i32 = jnp.int32
    edges = (ei_101[0].astype(i32), ei_101[1].astype(i32),
             ei_021[0].astype(i32), ei_021[1].astype(i32),
             ei_110[0].astype(i32), ei_110[1].astype(i32),
             ei_030[0].astype(i32), ei_030[1].astype(i32))
    h0, h1 = _layer(x0, x1, edges,
                    gin0_w1, gin0_b1, gin0_w2, gin0_b2, hl0_w, hl0_b,
                    bn0_g, bn0_b, final=False)
    return _layer(h0, h1, edges,
                  gin1_w1, gin1_b1, gin1_w2, gin1_b2, hl1_w, hl1_b,
                  bn1_g, bn1_b, final=True)
